# baseline (speedup 1.0000x reference)
import jax
import jax.numpy as jnp
from jax import lax
from jax.experimental import pallas as pl
from jax.experimental.pallas import tpu as pltpu

S = 256
A = 256
W = 8
D = 64
AW = A * W
SW = S * W
G = 128
NG = (AW + SW) // G + 1
N = NG * G
_CHUNK = 1024


def _issue_group(idx_s, table_ref, blocks, sems, gg, b):
    for j in range(G):
        r = idx_s[gg, j]
        blk = pl.multiple_of((r // 128) * 128, 128)
        pltpu.make_async_copy(
            table_ref.at[:, pl.ds(blk, 128)],
            blocks.at[b, pl.ds(D * j, D), :],
            sems.at[b],
        ).start()


def _wait_group(table_ref, blocks, sems, b):
    for _ in range(G):
        pltpu.make_async_copy(
            table_ref.at[:, pl.ds(0, 128)],
            blocks.at[b, pl.ds(0, D), :],
            sems.at[b],
        ).wait()


def _fused_body(idx_s, idx_v, table_ref, sims_ref, swe_ref,
                blocks, gcols, sems):
    f32 = jnp.float32
    i32 = jnp.int32
    g = pl.program_id(0)
    b = lax.rem(g, 2)

    @pl.when(g == 0)
    def _():
        _issue_group(idx_s, table_ref, blocks, sems, 0, 0)

    @pl.when(g + 1 < NG)
    def _():
        _issue_group(idx_s, table_ref, blocks, sems, g + 1, lax.rem(g + 1, 2))

    _wait_group(table_ref, blocks, sems, b)

    iv = idx_v[pl.ds(g, 1), :]
    mpos = lax.rem(iv, i32(128))
    onehot = jnp.where(
        lax.broadcasted_iota(i32, (128, G), 0) == mpos, f32(1.0), f32(0.0))
    esel = jnp.where(
        lax.broadcasted_iota(i32, (_CHUNK, D), 0) % D
        == lax.broadcasted_iota(i32, (_CHUNK, D), 1), f32(1.0), f32(0.0))
    c = jnp.zeros((D, G), f32)
    for t in range(D * G // _CHUNK):
        bt = blocks[b, pl.ds(t * _CHUNK, _CHUNK), :]
        pt = lax.dot_general(bt, onehot, (((1,), (0,)), ((), ())),
                             preferred_element_type=f32, precision=lax.Precision.HIGHEST)
        dmask = jnp.where(
            lax.broadcasted_iota(i32, (_CHUNK, G), 1)
            == lax.broadcasted_iota(i32, (_CHUNK, G), 0) // D
            + i32(t * (_CHUNK // D)), f32(1.0), f32(0.0))
        c = c + lax.dot_general(esel, pt * dmask, (((0,), (0,)), ((), ())),
                                preferred_element_type=f32, precision=lax.Precision.HIGHEST)
    gcols[:, pl.ds(pl.multiple_of(g * G, 128), G)] = c

    @pl.when(g == NG - 1)
    def _():
        cdims = (((0,), (0,)), ((), ()))
        ann = gcols[:, 0:AW]
        sen = gcols[:, AW:AW + SW]
        none_col = gcols[:, AW + SW:AW + SW + 1]
        ones_col = jnp.ones((D, 1), f32)
        an_row = jnp.sqrt(lax.dot_general(
            ones_col, ann * ann, cdims, preferred_element_type=f32, precision=lax.Precision.HIGHEST))
        sn_col = jnp.sqrt(lax.dot_general(
            sen * sen, ones_col, cdims, preferred_element_type=f32, precision=lax.Precision.HIGHEST))
        eqf = jnp.where(sen == none_col, 1.0, 0.0)
        eqcnt = lax.dot_general(eqf, ones_col, cdims,
                                preferred_element_type=f32, precision=lax.Precision.HIGHEST)
        isnone_col = eqcnt == f32(D)
        eye = jnp.where(
            lax.broadcasted_iota(i32, (D, D), 0)
            == lax.broadcasted_iota(i32, (D, D), 1), f32(1.0), f32(0.0))
        m = jnp.zeros((S, A), f32)
        swe = jnp.zeros((S, D), f32)
        for ws in range(W):
            sen_ws = sen[:, ws * S:(ws + 1) * S]
            swe = swe + lax.dot_general(sen_ws, eye, cdims,
                                        preferred_element_type=f32, precision=lax.Precision.HIGHEST)
            sn_c = sn_col[ws * S:(ws + 1) * S, :]
            isnone_c = isnone_col[ws * S:(ws + 1) * S, :]
            d_ws = lax.dot_general(sen_ws, ann, cdims,
                                   preferred_element_type=f32, precision=lax.Precision.HIGHEST)
            for wa in range(W):
                dvals = d_ws[:, wa * A:(wa + 1) * A]
                an_r = an_row[:, wa * A:(wa + 1) * A]
                denom = jnp.maximum(sn_c * an_r, 1e-8)
                sval = dvals / denom
                sval = jnp.where(isnone_c, 0.0, sval)
                m = jnp.where((sval >= m) | (sval < 0.0), sval, m)
        sims_ref[:] = m
        swe_ref[:] = swe


_fused = pl.pallas_call(
    _fused_body,
    grid=(NG,),
    in_specs=[
        pl.BlockSpec(memory_space=pltpu.SMEM),
        pl.BlockSpec(memory_space=pltpu.VMEM),
        pl.BlockSpec(memory_space=pltpu.HBM),
    ],
    out_specs=[
        pl.BlockSpec((S, A), lambda g: (0, 0)),
        pl.BlockSpec((S, D), lambda g: (0, 0)),
    ],
    out_shape=[
        jax.ShapeDtypeStruct((S, A), jnp.float32),
        jax.ShapeDtypeStruct((S, D), jnp.float32),
    ],
    scratch_shapes=[
        pltpu.VMEM((2, D * G, 128), jnp.float32),
        pltpu.VMEM((D, N), jnp.float32),
        pltpu.SemaphoreType.DMA((2,)),
    ],
)


def kernel(ann_cats, sen_cats, none_idx, table):
    ann_t = ann_cats.astype(jnp.int32).T.reshape(-1)
    sen_t = sen_cats.astype(jnp.int32).T.reshape(-1)
    nn = jnp.broadcast_to(none_idx.astype(jnp.int32), (G,))
    idx2d = jnp.concatenate([ann_t, sen_t, nn], axis=0).reshape(NG, G)
    table_t = table.T
    max_sims, swe = _fused(idx2d, idx2d, table_t)
    return max_sims, swe


# speedup vs baseline: 2.9291x; 2.9291x over previous
import jax
import jax.numpy as jnp
from jax import lax
from jax.experimental import pallas as pl
from jax.experimental.pallas import tpu as pltpu

S = 256
A = 256
W = 8
D = 64
AW = A * W
SW = S * W
G = 128
NG = (AW + SW) // G + 1
N = NG * G
_CHUNK = 1024


def _issue_group(idx_s, table_ref, blocks, sems, gg, b):
    for j in range(G):
        r = idx_s[gg, j]
        blk = pl.multiple_of((r // 128) * 128, 128)
        pltpu.make_async_copy(
            table_ref.at[:, pl.ds(blk, 128)],
            blocks.at[b, pl.ds(D * j, D), :],
            sems.at[b],
        ).start()


def _wait_group(table_ref, blocks, sems, b):
    for _ in range(G):
        pltpu.make_async_copy(
            table_ref.at[:, pl.ds(0, 128)],
            blocks.at[b, pl.ds(0, D), :],
            sems.at[b],
        ).wait()


def _fused_body(idx_s, idx_v, table_ref, sims_ref, swe_ref,
                blocks, gcols, sems):
    f32 = jnp.float32
    i32 = jnp.int32
    g = pl.program_id(0)
    b = lax.rem(g, 2)

    @pl.when(g == 0)
    def _():
        _issue_group(idx_s, table_ref, blocks, sems, 0, 0)

    @pl.when(g + 1 < NG)
    def _():
        _issue_group(idx_s, table_ref, blocks, sems, g + 1, lax.rem(g + 1, 2))

    _wait_group(table_ref, blocks, sems, b)

    iv = idx_v[pl.ds(g, 1), :]
    mpos = lax.rem(iv, i32(128))
    onehot = jnp.where(
        lax.broadcasted_iota(i32, (128, G), 0) == mpos, f32(1.0), f32(0.0))
    esel = jnp.where(
        lax.broadcasted_iota(i32, (_CHUNK, D), 0) % D
        == lax.broadcasted_iota(i32, (_CHUNK, D), 1), f32(1.0), f32(0.0))
    c = jnp.zeros((D, G), f32)
    for t in range(D * G // _CHUNK):
        bt = blocks[b, pl.ds(t * _CHUNK, _CHUNK), :]
        pt = lax.dot_general(bt, onehot, (((1,), (0,)), ((), ())),
                             preferred_element_type=f32)
        dmask = jnp.where(
            lax.broadcasted_iota(i32, (_CHUNK, G), 1)
            == lax.broadcasted_iota(i32, (_CHUNK, G), 0) // D
            + i32(t * (_CHUNK // D)), f32(1.0), f32(0.0))
        c = c + lax.dot_general(esel, pt * dmask, (((0,), (0,)), ((), ())),
                                preferred_element_type=f32)
    gcols[:, pl.ds(pl.multiple_of(g * G, 128), G)] = c

    @pl.when(g == NG - 1)
    def _():
        cdims = (((0,), (0,)), ((), ()))
        ann = gcols[:, 0:AW]
        sen = gcols[:, AW:AW + SW]
        none_col = gcols[:, AW + SW:AW + SW + 1]
        ones_col = jnp.ones((D, 1), f32)
        an_row = jnp.sqrt(lax.dot_general(
            ones_col, ann * ann, cdims, preferred_element_type=f32))
        sn_col = jnp.sqrt(lax.dot_general(
            sen * sen, ones_col, cdims, preferred_element_type=f32))
        eqf = jnp.where(sen == none_col, 1.0, 0.0)
        eqcnt = lax.dot_general(eqf, ones_col, cdims,
                                preferred_element_type=f32)
        isnone_col = eqcnt == f32(D)
        m = jnp.zeros((S, A), f32)
        swe = jnp.zeros((S, D), f32)
        for ws in range(W):
            sen_ws = sen[:, ws * S:(ws + 1) * S]
            swe = swe + jnp.transpose(sen_ws)
            sn_c = sn_col[ws * S:(ws + 1) * S, :]
            isnone_c = isnone_col[ws * S:(ws + 1) * S, :]
            d_ws = lax.dot_general(sen_ws, ann, cdims,
                                   preferred_element_type=f32)
            for wa in range(W):
                dvals = d_ws[:, wa * A:(wa + 1) * A]
                an_r = an_row[:, wa * A:(wa + 1) * A]
                denom = jnp.maximum(sn_c * an_r, 1e-8)
                sval = dvals / denom
                sval = jnp.where(isnone_c, 0.0, sval)
                m = jnp.where((sval >= m) | (sval < 0.0), sval, m)
        sims_ref[:] = m
        swe_ref[:] = swe


_fused = pl.pallas_call(
    _fused_body,
    grid=(NG,),
    in_specs=[
        pl.BlockSpec(memory_space=pltpu.SMEM),
        pl.BlockSpec(memory_space=pltpu.VMEM),
        pl.BlockSpec(memory_space=pltpu.HBM),
    ],
    out_specs=[
        pl.BlockSpec((S, A), lambda g: (0, 0)),
        pl.BlockSpec((S, D), lambda g: (0, 0)),
    ],
    out_shape=[
        jax.ShapeDtypeStruct((S, A), jnp.float32),
        jax.ShapeDtypeStruct((S, D), jnp.float32),
    ],
    scratch_shapes=[
        pltpu.VMEM((2, D * G, 128), jnp.float32),
        pltpu.VMEM((D, N), jnp.float32),
        pltpu.SemaphoreType.DMA((2,)),
    ],
)


def kernel(ann_cats, sen_cats, none_idx, table):
    ann_t = ann_cats.astype(jnp.int32).T.reshape(-1)
    sen_t = sen_cats.astype(jnp.int32).T.reshape(-1)
    nn = jnp.broadcast_to(none_idx.astype(jnp.int32), (G,))
    idx2d = jnp.concatenate([ann_t, sen_t, nn], axis=0).reshape(NG, G)
    table_t = table.T
    max_sims, swe = _fused(idx2d, idx2d, table_t)
    return max_sims, swe


# speedup vs baseline: 3.2929x; 1.1242x over previous
import jax
import jax.numpy as jnp
from jax import lax
from jax.experimental import pallas as pl
from jax.experimental.pallas import tpu as pltpu

S = 256
A = 256
W = 8
D = 64
AW = A * W
SW = S * W
G = 128
NG = (AW + SW) // G + 1
N = NG * G
_CHUNK = 1024


def _issue_group(idx_s, table_ref, blocks, sems, gg, b):
    for j in range(G):
        r = idx_s[gg, j]
        blk = pl.multiple_of((r // 128) * 128, 128)
        pltpu.make_async_copy(
            table_ref.at[:, pl.ds(blk, 128)],
            blocks.at[b, pl.ds(D * j, D), :],
            sems.at[b],
        ).start()


def _wait_group(table_ref, blocks, sems, b):
    for _ in range(G):
        pltpu.make_async_copy(
            table_ref.at[:, pl.ds(0, 128)],
            blocks.at[b, pl.ds(0, D), :],
            sems.at[b],
        ).wait()


def _fused_body(idx_s, idx_v, table_ref, sims_ref, swe_ref,
                blocks, gcols, emask, dmask, sems):
    f32 = jnp.float32
    i32 = jnp.int32
    half = D * G // 2
    g = pl.program_id(0)
    b = lax.rem(g, 2)

    @pl.when(g == 0)
    def _():
        emask[:] = jnp.where(
            lax.broadcasted_iota(i32, (half, D), 0) % D
            == lax.broadcasted_iota(i32, (half, D), 1), f32(1.0), f32(0.0))
        dmask[:] = jnp.where(
            lax.broadcasted_iota(i32, (half, G // 2), 0) // D
            == lax.broadcasted_iota(i32, (half, G // 2), 1), f32(1.0),
            f32(0.0))
        _issue_group(idx_s, table_ref, blocks, sems, 0, 0)

    @pl.when(g + 1 < NG)
    def _():
        _issue_group(idx_s, table_ref, blocks, sems, g + 1, lax.rem(g + 1, 2))

    _wait_group(table_ref, blocks, sems, b)

    iv = idx_v[pl.ds(g, 1), :]
    mpos = lax.rem(iv, i32(128))
    onehot = jnp.where(
        lax.broadcasted_iota(i32, (128, G), 0) == mpos, f32(1.0), f32(0.0))
    halves = []
    for h in range(2):
        bt = blocks[b, pl.ds(h * half, half), :]
        oh = onehot[:, h * (G // 2):(h + 1) * (G // 2)]
        pt = lax.dot_general(bt, oh, (((1,), (0,)), ((), ())),
                             preferred_element_type=f32)
        halves.append(lax.dot_general(
            emask[:], pt * dmask[:], (((0,), (0,)), ((), ())),
            preferred_element_type=f32))
    c = jnp.concatenate(halves, axis=1)
    gcols[:, pl.ds(pl.multiple_of(g * G, 128), G)] = c

    @pl.when(g == NG - 1)
    def _():
        cdims = (((0,), (0,)), ((), ()))
        ann = gcols[:, 0:AW]
        sen = gcols[:, AW:AW + SW]
        none_col = gcols[:, AW + SW:AW + SW + 1]
        ones_col = jnp.ones((D, 1), f32)
        an_row = jnp.sqrt(lax.dot_general(
            ones_col, ann * ann, cdims, preferred_element_type=f32))
        sn_col = jnp.sqrt(lax.dot_general(
            sen * sen, ones_col, cdims, preferred_element_type=f32))
        eqf = jnp.where(sen == none_col, 1.0, 0.0)
        eqcnt = lax.dot_general(eqf, ones_col, cdims,
                                preferred_element_type=f32)
        isnone_col = eqcnt == f32(D)
        m = jnp.zeros((S, A), f32)
        swe = jnp.zeros((S, D), f32)
        for ws in range(W):
            sen_ws = sen[:, ws * S:(ws + 1) * S]
            swe = swe + jnp.transpose(sen_ws)
            sn_c = sn_col[ws * S:(ws + 1) * S, :]
            isnone_c = isnone_col[ws * S:(ws + 1) * S, :]
            d_ws = lax.dot_general(sen_ws, ann, cdims,
                                   preferred_element_type=f32)
            for wa in range(W):
                dvals = d_ws[:, wa * A:(wa + 1) * A]
                an_r = an_row[:, wa * A:(wa + 1) * A]
                denom = jnp.maximum(sn_c * an_r, 1e-8)
                sval = dvals / denom
                sval = jnp.where(isnone_c, 0.0, sval)
                m = jnp.where((sval >= m) | (sval < 0.0), sval, m)
        sims_ref[:] = m
        swe_ref[:] = swe


_fused = pl.pallas_call(
    _fused_body,
    grid=(NG,),
    in_specs=[
        pl.BlockSpec(memory_space=pltpu.SMEM),
        pl.BlockSpec(memory_space=pltpu.VMEM),
        pl.BlockSpec(memory_space=pltpu.HBM),
    ],
    out_specs=[
        pl.BlockSpec((S, A), lambda g: (0, 0)),
        pl.BlockSpec((S, D), lambda g: (0, 0)),
    ],
    out_shape=[
        jax.ShapeDtypeStruct((S, A), jnp.float32),
        jax.ShapeDtypeStruct((S, D), jnp.float32),
    ],
    scratch_shapes=[
        pltpu.VMEM((2, D * G, 128), jnp.float32),
        pltpu.VMEM((D, N), jnp.float32),
        pltpu.VMEM((D * G // 2, D), jnp.float32),
        pltpu.VMEM((D * G // 2, G // 2), jnp.float32),
        pltpu.SemaphoreType.DMA((2,)),
    ],
)


def kernel(ann_cats, sen_cats, none_idx, table):
    ann_t = ann_cats.astype(jnp.int32).T.reshape(-1)
    sen_t = sen_cats.astype(jnp.int32).T.reshape(-1)
    nn = jnp.broadcast_to(none_idx.astype(jnp.int32), (G,))
    idx2d = jnp.concatenate([ann_t, sen_t, nn], axis=0).reshape(NG, G)
    table_t = table.T
    max_sims, swe = _fused(idx2d, idx2d, table_t)
    return max_sims, swe
